# baseline (speedup 1.0000x reference)
import functools

import jax
import jax.numpy as jnp
from jax import lax
from jax.experimental import pallas as pl
from jax.experimental.pallas import tpu as pltpu
from jax.experimental.pallas import tpu_sc as plsc

N = 10000
H = 128
R = 7
E = 320000

EP = 327680
EROWS = EP // 128
RN = R * N
RNPAD = 71680

_MESH = plsc.VectorSubcoreMesh(
    core_axis_name="c", subcore_axis_name="s", num_cores=2, num_subcores=16)

def _z16():
    return jnp.zeros((16,), jnp.float32)


@functools.partial(
    pl.kernel,
    out_type=jax.ShapeDtypeStruct((RNPAD,), jnp.float32),
    mesh=_MESH,
    scratch_types=[
        pltpu.VMEM_SHARED((RNPAD,), jnp.float32),
        pltpu.VMEM((8, 128), jnp.int32),
        pltpu.VMEM((128,), jnp.float32),
        pltpu.VMEM((4480,), jnp.float32),
    ],
)
def _counts_inv(sidx_hbm, inv_hbm, hist, sv, ones_v, loc):
    c = lax.axis_index("c")
    s = lax.axis_index("s")

    def _zero(i, _):
        loc[pl.ds(i * 16, 16)] = _z16()
        return 0
    lax.fori_loop(0, 280, _zero, 0)
    pltpu.sync_copy(loc, hist.at[pl.ds(s * 4480, 4480)])
    for i in range(8):
        ones_v[pl.ds(i * 16, 16)] = jnp.ones((16,), jnp.float32)
    plsc.subcore_barrier()

    def _histo(t, _):
        rb = s * 160 + t * 8
        pltpu.sync_copy(sidx_hbm.at[pl.ds(rb, 8)], sv)
        for j in range(8):
            pltpu.sync_copy(ones_v, hist.at[sv.at[j]], add=True)
        return 0
    lax.fori_loop(0, 20, _histo, 0)
    plsc.subcore_barrier()

    pltpu.sync_copy(hist.at[pl.ds(s * 4480, 4480)], loc)

    def _inv(i, _):
        v = loc[pl.ds(i * 16, 16)]
        g = s * 4480 + i * 16 + lax.iota(jnp.int32, 16)
        r = 1.0 / jnp.maximum(v, 1.0)
        loc[pl.ds(i * 16, 16)] = jnp.where(g < RN, r, 0.0)
        return 0
    lax.fori_loop(0, 280, _inv, 0)

    @pl.when(c == 0)
    def _():
        pltpu.sync_copy(loc, inv_hbm.at[pl.ds(s * 4480, 4480)])


@functools.partial(
    pl.kernel,
    out_type=jax.ShapeDtypeStruct((2, N, H), jnp.float32),
    mesh=_MESH,
    scratch_types=[
        pltpu.VMEM_SHARED((N, H), jnp.float32),
        pltpu.VMEM((2, 128), jnp.int32),
        pltpu.VMEM((2, 128), jnp.int32),
        pltpu.VMEM((2, 128), jnp.int32),
        pltpu.VMEM((256, H), jnp.float32),
        pltpu.VMEM((256,), jnp.float32),
        pltpu.SemaphoreType.DMA,
        pltpu.SemaphoreType.DMA,
    ],
)
def _edge_pass(gidx_hbm, sidx_hbm, dst_hbm, xw_hbm, inv_hbm, accs_hbm,
               acc, gi, si, dv, rows, sc_v, gsem, ssem):
    c = lax.axis_index("c")
    s = lax.axis_index("s")
    start_row = jnp.where(c == 0, s * 138, 2208 + s * 22)
    n_chunks = jnp.where(c == 0, 69, 11)

    def _zero(i, _):
        for j in range(8):
            rows[i, pl.ds(j * 16, 16)] = _z16()
        return 0
    lax.fori_loop(0, 256, _zero, 0)
    base = s * 624
    pltpu.sync_copy(rows.at[pl.ds(0, 256)], acc.at[pl.ds(base, 256)])
    pltpu.sync_copy(rows.at[pl.ds(0, 256)], acc.at[pl.ds(base + 256, 256)])
    pltpu.sync_copy(rows.at[pl.ds(0, 112)], acc.at[pl.ds(base + 512, 112)])

    @pl.when(s == 0)
    def _():
        pltpu.sync_copy(rows.at[pl.ds(0, 16)], acc.at[pl.ds(9984, 16)])
    plsc.subcore_barrier()

    def _chunk(t, _):
        rb = start_row + t * 2
        pltpu.sync_copy(gidx_hbm.at[pl.ds(rb, 2)], gi)
        pltpu.sync_copy(sidx_hbm.at[pl.ds(rb, 2)], si)
        pltpu.sync_copy(dst_hbm.at[pl.ds(rb, 2)], dv)
        cps = []
        for j in range(2):
            cps.append(pltpu.async_copy(
                xw_hbm.at[gi.at[j]], rows.at[pl.ds(j * 128, 128)], gsem))
        for j in range(2):
            cps.append(pltpu.async_copy(
                inv_hbm.at[si.at[j]], sc_v.at[pl.ds(j * 128, 128)], ssem))
        for cp in cps:
            cp.wait()

        def _scale(k, _):
            sv16 = sc_v[pl.ds(k * 16, 16)]
            for l in range(16):
                sval = sv16[l]
                i = k * 16 + l
                for j in range(8):
                    rows[i, pl.ds(j * 16, 16)] = (
                        rows[i, pl.ds(j * 16, 16)] * sval)
            return 0
        lax.fori_loop(0, 16, _scale, 0)

        for j in range(2):
            pltpu.sync_copy(rows.at[pl.ds(j * 128, 128)],
                            acc.at[dv.at[j]], add=True)
        return 0
    lax.fori_loop(0, n_chunks, _chunk, 0)
    plsc.subcore_barrier()

    cps = []
    for k, nrow in [(i * 40, 40) for i in range(15)] + [(600, 24)]:
        cps.append(pltpu.async_copy(
            acc.at[pl.ds(base + k, nrow)],
            accs_hbm.at[c, pl.ds(base + k, nrow)], gsem))
    for cp in cps:
        cp.wait()

    @pl.when(s == 0)
    def _():
        pltpu.sync_copy(acc.at[pl.ds(9984, 16)],
                        accs_hbm.at[c, pl.ds(9984, 16)])


_BN = 1000


def _mm8_body(x_ref, t_ref, o_ref):
    o_ref[0] = jnp.dot(x_ref[...], t_ref[0], preferred_element_type=jnp.float32)


_mm8 = pl.pallas_call(
    _mm8_body,
    grid=(8, N // _BN),
    in_specs=[
        pl.BlockSpec((_BN, H), lambda r, j: (j, 0)),
        pl.BlockSpec((1, H, H), lambda r, j: (r, 0, 0)),
    ],
    out_specs=pl.BlockSpec((1, _BN, H), lambda r, j: (r, j, 0)),
    out_shape=jax.ShapeDtypeStruct((8, N, H), jnp.float32),
)


def _combine_body(x_ref, a_ref, b_ref, bias_ref, o_ref):
    o_ref[...] = jax.nn.relu(
        x_ref[...] + a_ref[...] + b_ref[...] + bias_ref[...])


_combine = pl.pallas_call(
    _combine_body,
    grid=(N // _BN,),
    in_specs=[
        pl.BlockSpec((_BN, H), lambda j: (j, 0)),
        pl.BlockSpec((_BN, H), lambda j: (j, 0)),
        pl.BlockSpec((_BN, H), lambda j: (j, 0)),
        pl.BlockSpec((1, H), lambda j: (0, 0)),
    ],
    out_specs=pl.BlockSpec((_BN, H), lambda j: (j, 0)),
    out_shape=jax.ShapeDtypeStruct((N, H), jnp.float32),
)


def _head_body(x_ref, a_ref, b_ref, bias_ref, wh_ref, bh_ref, o_ref):
    h = jax.nn.relu(x_ref[...] + a_ref[...] + b_ref[...] + bias_ref[...])
    o_ref[...] = jnp.dot(h, wh_ref[...],
                         preferred_element_type=jnp.float32) + bh_ref[...]


_head = pl.pallas_call(
    _head_body,
    grid=(N // _BN,),
    in_specs=[
        pl.BlockSpec((_BN, H), lambda j: (j, 0)),
        pl.BlockSpec((_BN, H), lambda j: (j, 0)),
        pl.BlockSpec((_BN, H), lambda j: (j, 0)),
        pl.BlockSpec((1, H), lambda j: (0, 0)),
        pl.BlockSpec((H, 8), lambda j: (0, 0)),
        pl.BlockSpec((1, 8), lambda j: (0, 0)),
    ],
    out_specs=pl.BlockSpec((_BN, 8), lambda j: (j, 0)),
    out_shape=jax.ShapeDtypeStruct((N, 8), jnp.float32),
)


def kernel(x, edge_index, edge_type, edge_weight, W1, root1, b1,
           W2, root2, b2, bot_w, bot_b, stance_w, stance_b):
    del edge_weight
    src = edge_index[0].astype(jnp.int32)
    dst = edge_index[1].astype(jnp.int32)
    et = edge_type.astype(jnp.int32)

    gidx = et * N + src
    sidx = et * N + dst
    npad = EP - E
    gidx = jnp.concatenate([gidx, jnp.full((npad,), R * N, jnp.int32)])
    sidx = jnp.concatenate([sidx, jnp.full((npad,), RN, jnp.int32)])
    dstp = jnp.concatenate([dst, jnp.zeros((npad,), jnp.int32)])
    gidx2 = gidx.reshape(EROWS, 128)
    sidx2 = sidx.reshape(EROWS, 128)
    dst2 = dstp.reshape(EROWS, 128)

    inv = _counts_inv(sidx2)

    T1 = jnp.concatenate([W1, root1[None]], axis=0)
    T2 = jnp.concatenate([W2, root2[None]], axis=0)
    wh = jnp.concatenate(
        [bot_w, stance_w, jnp.zeros((H, 4), jnp.float32)], axis=1)
    bh = jnp.concatenate(
        [bot_b, stance_b, jnp.zeros((4,), jnp.float32)])[None]

    xw1 = _mm8(x, T1)
    accs1 = _edge_pass(gidx2, sidx2, dst2, xw1.reshape(8 * N, H), inv)
    h = _combine(xw1[7], accs1[0], accs1[1], b1[None])

    xw2 = _mm8(h, T2)
    accs2 = _edge_pass(gidx2, sidx2, dst2, xw2.reshape(8 * N, H), inv)
    out = _head(xw2[7], accs2[0], accs2[1], b2[None], wh, bh)

    return (out[:, 0], out[:, 1:4])


# speedup vs baseline: 1.1285x; 1.1285x over previous
import functools

import jax
import jax.numpy as jnp
from jax import lax
from jax.experimental import pallas as pl
from jax.experimental.pallas import tpu as pltpu
from jax.experimental.pallas import tpu_sc as plsc

N = 10000
H = 128
R = 7
E = 320000

EP = 327680
EROWS = EP // 128
RN = R * N
RNPAD = 71680

_MESH = plsc.VectorSubcoreMesh(
    core_axis_name="c", subcore_axis_name="s", num_cores=2, num_subcores=16)

def _z16():
    return jnp.zeros((16,), jnp.float32)


@functools.partial(
    pl.kernel,
    out_type=jax.ShapeDtypeStruct((RNPAD,), jnp.float32),
    mesh=_MESH,
    scratch_types=[
        pltpu.VMEM_SHARED((RNPAD,), jnp.float32),
        pltpu.VMEM((8, 128), jnp.int32),
        pltpu.VMEM((128,), jnp.float32),
        pltpu.VMEM((4480,), jnp.float32),
    ],
)
def _counts_inv(sidx_hbm, inv_hbm, hist, sv, ones_v, loc):
    c = lax.axis_index("c")
    s = lax.axis_index("s")

    def _zero(i, _):
        loc[pl.ds(i * 16, 16)] = _z16()
        return 0
    lax.fori_loop(0, 280, _zero, 0)
    pltpu.sync_copy(loc, hist.at[pl.ds(s * 4480, 4480)])
    for i in range(8):
        ones_v[pl.ds(i * 16, 16)] = jnp.ones((16,), jnp.float32)
    plsc.subcore_barrier()

    def _histo(t, _):
        rb = s * 160 + t * 8
        pltpu.sync_copy(sidx_hbm.at[pl.ds(rb, 8)], sv)
        for j in range(8):
            pltpu.sync_copy(ones_v, hist.at[sv.at[j]], add=True)
        return 0
    lax.fori_loop(0, 20, _histo, 0)
    plsc.subcore_barrier()

    pltpu.sync_copy(hist.at[pl.ds(s * 4480, 4480)], loc)

    def _inv(i, _):
        v = loc[pl.ds(i * 16, 16)]
        g = s * 4480 + i * 16 + lax.iota(jnp.int32, 16)
        r = 1.0 / jnp.maximum(v, 1.0)
        loc[pl.ds(i * 16, 16)] = jnp.where(g < RN, r, 0.0)
        return 0
    lax.fori_loop(0, 280, _inv, 0)

    @pl.when(c == 0)
    def _():
        pltpu.sync_copy(loc, inv_hbm.at[pl.ds(s * 4480, 4480)])


@functools.partial(
    pl.kernel,
    out_type=jax.ShapeDtypeStruct((2, N, H), jnp.float32),
    mesh=_MESH,
    scratch_types=[
        pltpu.VMEM_SHARED((N, H), jnp.float32),
        pltpu.VMEM((16, 128), jnp.int32),
        pltpu.VMEM((16, 128), jnp.int32),
        pltpu.VMEM((16, 128), jnp.int32),
        pltpu.VMEM((128, H), jnp.float32),
        pltpu.VMEM((128, H), jnp.float32),
        pltpu.VMEM((128,), jnp.float32),
        pltpu.VMEM((128,), jnp.float32),
        pltpu.SemaphoreType.DMA,
        pltpu.SemaphoreType.DMA,
        pltpu.SemaphoreType.DMA,
        pltpu.SemaphoreType.DMA,
    ],
)
def _edge_pass(gidx_hbm, sidx_hbm, dst_hbm, xw_hbm, inv_hbm, accs_hbm,
               acc, gi, si, dv, rows0, rows1, sc0, sc1, g0, g1, w0, w1):
    c = lax.axis_index("c")
    s = lax.axis_index("s")
    start_row = jnp.where(c == 0, s * 144, 2304 + s * 16)
    nblocks = jnp.where(c == 0, 9, 1)
    ROWS = (rows0, rows1)
    SCB = (sc0, sc1)
    GS = (g0, g1)
    WS = (w0, w1)

    def _zero(i, _):
        for j in range(8):
            rows0[i, pl.ds(j * 16, 16)] = _z16()
        return 0
    lax.fori_loop(0, 128, _zero, 0)
    base = s * 624
    for k in range(4):
        pltpu.sync_copy(rows0, acc.at[pl.ds(base + k * 128, 128)])
    pltpu.sync_copy(rows0.at[pl.ds(0, 112)], acc.at[pl.ds(base + 512, 112)])

    @pl.when(s == 0)
    def _():
        pltpu.sync_copy(rows0.at[pl.ds(0, 16)], acc.at[pl.ds(9984, 16)])
    plsc.subcore_barrier()

    def _fire(u):
        p = u % 2
        return (pltpu.async_copy(xw_hbm.at[gi.at[u]], ROWS[p], GS[p]),
                pltpu.async_copy(inv_hbm.at[si.at[u]], SCB[p], GS[p]))

    def _block(t, _):
        rb = start_row + t * 16
        pltpu.sync_copy(gidx_hbm.at[pl.ds(rb, 16)], gi)
        pltpu.sync_copy(sidx_hbm.at[pl.ds(rb, 16)], si)
        pltpu.sync_copy(dst_hbm.at[pl.ds(rb, 16)], dv)
        gcps = {0: _fire(0)}
        wcps = {}
        for u in range(16):
            p = u % 2
            if u < 15:
                if u >= 1:
                    wcps[u - 1].wait()
                gcps[u + 1] = _fire(u + 1)
            for cp in gcps[u]:
                cp.wait()

            def _scale(k, _, p=p):
                sv16 = SCB[p][pl.ds(k * 16, 16)]
                for l in range(16):
                    sval = sv16[l]
                    i = k * 16 + l
                    for j in range(8):
                        ROWS[p][i, pl.ds(j * 16, 16)] = (
                            ROWS[p][i, pl.ds(j * 16, 16)] * sval)
                return 0
            lax.fori_loop(0, 8, _scale, 0)

            wcps[u] = pltpu.async_copy(ROWS[p], acc.at[dv.at[u]], WS[p],
                                       add=True)
        wcps[14].wait()
        wcps[15].wait()
        return 0
    lax.fori_loop(0, nblocks, _block, 0)
    plsc.subcore_barrier()

    cps = []
    for k, nrow in [(i * 40, 40) for i in range(15)] + [(600, 24)]:
        cps.append(pltpu.async_copy(
            acc.at[pl.ds(base + k, nrow)],
            accs_hbm.at[c, pl.ds(base + k, nrow)], g0))
    for cp in cps:
        cp.wait()


_BN = 1000


def _mm8_body(x_ref, t_ref, o_ref):
    o_ref[0] = jnp.dot(x_ref[...], t_ref[0], preferred_element_type=jnp.float32)


_mm8 = pl.pallas_call(
    _mm8_body,
    grid=(8, N // _BN),
    in_specs=[
        pl.BlockSpec((_BN, H), lambda r, j: (j, 0)),
        pl.BlockSpec((1, H, H), lambda r, j: (r, 0, 0)),
    ],
    out_specs=pl.BlockSpec((1, _BN, H), lambda r, j: (r, j, 0)),
    out_shape=jax.ShapeDtypeStruct((8, N, H), jnp.float32),
)


def _combine_body(x_ref, a_ref, b_ref, bias_ref, o_ref):
    o_ref[...] = jax.nn.relu(
        x_ref[...] + a_ref[...] + b_ref[...] + bias_ref[...])


_combine = pl.pallas_call(
    _combine_body,
    grid=(N // _BN,),
    in_specs=[
        pl.BlockSpec((_BN, H), lambda j: (j, 0)),
        pl.BlockSpec((_BN, H), lambda j: (j, 0)),
        pl.BlockSpec((_BN, H), lambda j: (j, 0)),
        pl.BlockSpec((1, H), lambda j: (0, 0)),
    ],
    out_specs=pl.BlockSpec((_BN, H), lambda j: (j, 0)),
    out_shape=jax.ShapeDtypeStruct((N, H), jnp.float32),
)


def _head_body(x_ref, a_ref, b_ref, bias_ref, wh_ref, bh_ref, o_ref):
    h = jax.nn.relu(x_ref[...] + a_ref[...] + b_ref[...] + bias_ref[...])
    o_ref[...] = jnp.dot(h, wh_ref[...],
                         preferred_element_type=jnp.float32) + bh_ref[...]


_head = pl.pallas_call(
    _head_body,
    grid=(N // _BN,),
    in_specs=[
        pl.BlockSpec((_BN, H), lambda j: (j, 0)),
        pl.BlockSpec((_BN, H), lambda j: (j, 0)),
        pl.BlockSpec((_BN, H), lambda j: (j, 0)),
        pl.BlockSpec((1, H), lambda j: (0, 0)),
        pl.BlockSpec((H, 8), lambda j: (0, 0)),
        pl.BlockSpec((1, 8), lambda j: (0, 0)),
    ],
    out_specs=pl.BlockSpec((_BN, 8), lambda j: (j, 0)),
    out_shape=jax.ShapeDtypeStruct((N, 8), jnp.float32),
)


def kernel(x, edge_index, edge_type, edge_weight, W1, root1, b1,
           W2, root2, b2, bot_w, bot_b, stance_w, stance_b):
    del edge_weight
    src = edge_index[0].astype(jnp.int32)
    dst = edge_index[1].astype(jnp.int32)
    et = edge_type.astype(jnp.int32)

    gidx = et * N + src
    sidx = et * N + dst
    npad = EP - E
    gidx = jnp.concatenate([gidx, jnp.full((npad,), R * N, jnp.int32)])
    sidx = jnp.concatenate([sidx, jnp.full((npad,), RN, jnp.int32)])
    dstp = jnp.concatenate([dst, jnp.zeros((npad,), jnp.int32)])
    gidx2 = gidx.reshape(EROWS, 128)
    sidx2 = sidx.reshape(EROWS, 128)
    dst2 = dstp.reshape(EROWS, 128)

    inv = _counts_inv(sidx2)

    T1 = jnp.concatenate([W1, root1[None]], axis=0)
    T2 = jnp.concatenate([W2, root2[None]], axis=0)
    wh = jnp.concatenate(
        [bot_w, stance_w, jnp.zeros((H, 4), jnp.float32)], axis=1)
    bh = jnp.concatenate(
        [bot_b, stance_b, jnp.zeros((4,), jnp.float32)])[None]

    xw1 = _mm8(x, T1)
    accs1 = _edge_pass(gidx2, sidx2, dst2, xw1.reshape(8 * N, H), inv)
    h = _combine(xw1[7], accs1[0], accs1[1], b1[None])

    xw2 = _mm8(h, T2)
    accs2 = _edge_pass(gidx2, sidx2, dst2, xw2.reshape(8 * N, H), inv)
    out = _head(xw2[7], accs2[0], accs2[1], b2[None], wh, bh)

    return (out[:, 0], out[:, 1:4])


# speedup vs baseline: 1.1302x; 1.0015x over previous
import functools

import jax
import jax.numpy as jnp
from jax import lax
from jax.experimental import pallas as pl
from jax.experimental.pallas import tpu as pltpu
from jax.experimental.pallas import tpu_sc as plsc

N = 10000
H = 128
R = 7
E = 320000

EP = 327680
EROWS = EP // 128
RN = R * N
RNPAD = 71680

_MESH = plsc.VectorSubcoreMesh(
    core_axis_name="c", subcore_axis_name="s", num_cores=2, num_subcores=16)

def _z16():
    return jnp.zeros((16,), jnp.float32)


@functools.partial(
    pl.kernel,
    out_type=jax.ShapeDtypeStruct((RNPAD,), jnp.float32),
    mesh=_MESH,
    scratch_types=[
        pltpu.VMEM_SHARED((RNPAD,), jnp.float32),
        pltpu.VMEM((8, 128), jnp.int32),
        pltpu.VMEM((128,), jnp.float32),
        pltpu.VMEM((4480,), jnp.float32),
    ],
)
def _counts_inv(sidx_hbm, inv_hbm, hist, sv, ones_v, loc):
    c = lax.axis_index("c")
    s = lax.axis_index("s")

    def _zero(i, _):
        loc[pl.ds(i * 16, 16)] = _z16()
        return 0
    lax.fori_loop(0, 280, _zero, 0)
    pltpu.sync_copy(loc, hist.at[pl.ds(s * 4480, 4480)])
    for i in range(8):
        ones_v[pl.ds(i * 16, 16)] = jnp.ones((16,), jnp.float32)
    plsc.subcore_barrier()

    def _histo(t, _):
        rb = s * 160 + t * 8
        pltpu.sync_copy(sidx_hbm.at[pl.ds(rb, 8)], sv)
        for j in range(8):
            pltpu.sync_copy(ones_v, hist.at[sv.at[j]], add=True)
        return 0
    lax.fori_loop(0, 20, _histo, 0)
    plsc.subcore_barrier()

    pltpu.sync_copy(hist.at[pl.ds(s * 4480, 4480)], loc)

    def _inv(i, _):
        v = loc[pl.ds(i * 16, 16)]
        g = s * 4480 + i * 16 + lax.iota(jnp.int32, 16)
        r = 1.0 / jnp.maximum(v, 1.0)
        loc[pl.ds(i * 16, 16)] = jnp.where(g < RN, r, 0.0)
        return 0
    lax.fori_loop(0, 280, _inv, 0)

    @pl.when(c == 0)
    def _():
        pltpu.sync_copy(loc, inv_hbm.at[pl.ds(s * 4480, 4480)])


@functools.partial(
    pl.kernel,
    out_type=jax.ShapeDtypeStruct((2, N, H), jnp.float32),
    mesh=_MESH,
    scratch_types=[
        pltpu.VMEM_SHARED((N, H), jnp.float32),
        pltpu.VMEM((16, 128), jnp.int32),
        pltpu.VMEM((16, 128), jnp.int32),
        pltpu.VMEM((16, 128), jnp.int32),
        pltpu.VMEM((128, H), jnp.float32),
        pltpu.VMEM((128, H), jnp.float32),
        pltpu.VMEM((128,), jnp.float32),
        pltpu.VMEM((128,), jnp.float32),
        pltpu.SemaphoreType.DMA,
        pltpu.SemaphoreType.DMA,
        pltpu.SemaphoreType.DMA,
        pltpu.SemaphoreType.DMA,
    ],
)
def _edge_pass(gidx_hbm, sidx_hbm, dst_hbm, xw_hbm, inv_hbm, accs_hbm,
               acc, gi, si, dv, rows0, rows1, sc0, sc1, g0, g1, w0, w1):
    c = lax.axis_index("c")
    s = lax.axis_index("s")
    start_row = jnp.where(c == 0, s * 144, 2304 + s * 16)
    nblocks = jnp.where(c == 0, 9, 1)
    ROWS = (rows0, rows1)
    SCB = (sc0, sc1)
    GS = (g0, g1)
    WS = (w0, w1)

    def _zero(i, _):
        for j in range(8):
            rows0[i, pl.ds(j * 16, 16)] = _z16()
        return 0
    lax.fori_loop(0, 128, _zero, 0)
    base = s * 624
    for k in range(4):
        pltpu.sync_copy(rows0, acc.at[pl.ds(base + k * 128, 128)])
    pltpu.sync_copy(rows0.at[pl.ds(0, 112)], acc.at[pl.ds(base + 512, 112)])

    @pl.when(s == 0)
    def _():
        pltpu.sync_copy(rows0.at[pl.ds(0, 16)], acc.at[pl.ds(9984, 16)])
    plsc.subcore_barrier()

    def _fire(u):
        p = u % 2
        return (pltpu.async_copy(xw_hbm.at[gi.at[u]], ROWS[p], GS[p]),
                pltpu.async_copy(inv_hbm.at[si.at[u]], SCB[p], GS[p]))

    def _block(t, _):
        rb = start_row + t * 16
        pltpu.sync_copy(gidx_hbm.at[pl.ds(rb, 16)], gi)
        pltpu.sync_copy(sidx_hbm.at[pl.ds(rb, 16)], si)
        pltpu.sync_copy(dst_hbm.at[pl.ds(rb, 16)], dv)
        gcps = {0: _fire(0)}
        for u in range(16):
            p = u % 2
            if u < 15:
                gcps[u + 1] = _fire(u + 1)
            for cp in gcps[u]:
                cp.wait()

            def _scale(k, _, p=p):
                sv16 = SCB[p][pl.ds(k * 16, 16)]
                for l in range(16):
                    sval = sv16[l]
                    i = k * 16 + l
                    for j in range(8):
                        ROWS[p][i, pl.ds(j * 16, 16)] = (
                            ROWS[p][i, pl.ds(j * 16, 16)] * sval)
                return 0
            lax.fori_loop(0, 8, _scale, 0)

            pltpu.sync_copy(ROWS[p], acc.at[dv.at[u]], add=True)
        return 0
    lax.fori_loop(0, nblocks, _block, 0)
    plsc.subcore_barrier()

    cps = []
    for k, nrow in [(i * 40, 40) for i in range(15)] + [(600, 24)]:
        cps.append(pltpu.async_copy(
            acc.at[pl.ds(base + k, nrow)],
            accs_hbm.at[c, pl.ds(base + k, nrow)], g0))
    for cp in cps:
        cp.wait()


_BN = 1000


def _mm8_body(x_ref, t_ref, o_ref):
    o_ref[0] = jnp.dot(x_ref[...], t_ref[0], preferred_element_type=jnp.float32)


_mm8 = pl.pallas_call(
    _mm8_body,
    grid=(8, N // _BN),
    in_specs=[
        pl.BlockSpec((_BN, H), lambda r, j: (j, 0)),
        pl.BlockSpec((1, H, H), lambda r, j: (r, 0, 0)),
    ],
    out_specs=pl.BlockSpec((1, _BN, H), lambda r, j: (r, j, 0)),
    out_shape=jax.ShapeDtypeStruct((8, N, H), jnp.float32),
)


def _combine_body(x_ref, a_ref, b_ref, bias_ref, o_ref):
    o_ref[...] = jax.nn.relu(
        x_ref[...] + a_ref[...] + b_ref[...] + bias_ref[...])


_combine = pl.pallas_call(
    _combine_body,
    grid=(N // _BN,),
    in_specs=[
        pl.BlockSpec((_BN, H), lambda j: (j, 0)),
        pl.BlockSpec((_BN, H), lambda j: (j, 0)),
        pl.BlockSpec((_BN, H), lambda j: (j, 0)),
        pl.BlockSpec((1, H), lambda j: (0, 0)),
    ],
    out_specs=pl.BlockSpec((_BN, H), lambda j: (j, 0)),
    out_shape=jax.ShapeDtypeStruct((N, H), jnp.float32),
)


def _head_body(x_ref, a_ref, b_ref, bias_ref, wh_ref, bh_ref, o_ref):
    h = jax.nn.relu(x_ref[...] + a_ref[...] + b_ref[...] + bias_ref[...])
    o_ref[...] = jnp.dot(h, wh_ref[...],
                         preferred_element_type=jnp.float32) + bh_ref[...]


_head = pl.pallas_call(
    _head_body,
    grid=(N // _BN,),
    in_specs=[
        pl.BlockSpec((_BN, H), lambda j: (j, 0)),
        pl.BlockSpec((_BN, H), lambda j: (j, 0)),
        pl.BlockSpec((_BN, H), lambda j: (j, 0)),
        pl.BlockSpec((1, H), lambda j: (0, 0)),
        pl.BlockSpec((H, 8), lambda j: (0, 0)),
        pl.BlockSpec((1, 8), lambda j: (0, 0)),
    ],
    out_specs=pl.BlockSpec((_BN, 8), lambda j: (j, 0)),
    out_shape=jax.ShapeDtypeStruct((N, 8), jnp.float32),
)


def kernel(x, edge_index, edge_type, edge_weight, W1, root1, b1,
           W2, root2, b2, bot_w, bot_b, stance_w, stance_b):
    del edge_weight
    src = edge_index[0].astype(jnp.int32)
    dst = edge_index[1].astype(jnp.int32)
    et = edge_type.astype(jnp.int32)

    gidx = et * N + src
    sidx = et * N + dst
    npad = EP - E
    gidx = jnp.concatenate([gidx, jnp.full((npad,), R * N, jnp.int32)])
    sidx = jnp.concatenate([sidx, jnp.full((npad,), RN, jnp.int32)])
    dstp = jnp.concatenate([dst, jnp.zeros((npad,), jnp.int32)])
    gidx2 = gidx.reshape(EROWS, 128)
    sidx2 = sidx.reshape(EROWS, 128)
    dst2 = dstp.reshape(EROWS, 128)

    inv = _counts_inv(sidx2)

    T1 = jnp.concatenate([W1, root1[None]], axis=0)
    T2 = jnp.concatenate([W2, root2[None]], axis=0)
    wh = jnp.concatenate(
        [bot_w, stance_w, jnp.zeros((H, 4), jnp.float32)], axis=1)
    bh = jnp.concatenate(
        [bot_b, stance_b, jnp.zeros((4,), jnp.float32)])[None]

    xw1 = _mm8(x, T1)
    accs1 = _edge_pass(gidx2, sidx2, dst2, xw1.reshape(8 * N, H), inv)
    h = _combine(xw1[7], accs1[0], accs1[1], b1[None])

    xw2 = _mm8(h, T2)
    accs2 = _edge_pass(gidx2, sidx2, dst2, xw2.reshape(8 * N, H), inv)
    out = _head(xw2[7], accs2[0], accs2[1], b2[None], wh, bh)

    return (out[:, 0], out[:, 1:4])
